# Initial kernel scaffold; baseline (speedup 1.0000x reference)
import jax
import jax.numpy as jnp
from jax.experimental import pallas as pl


def kernel(x, W, b):
    raise NotImplementedError("write your pallas kernel here")


# speedup vs baseline: 1.0620x; 1.0620x over previous
import jax
import jax.numpy as jnp
from jax.experimental import pallas as pl

N_TOK = 8192
D_MODEL = 4096
N_EXP = 64
TOP_K = 2
TM = 256


def _gate_body(x_ref, wt_ref, b_ref, tw_ref, ti_ref, w_ref):
    logits = jax.lax.dot_general(
        x_ref[...], wt_ref[...], (((1,), (0,)), ((), ())),
        preferred_element_type=jnp.float32,
        precision=jax.lax.Precision.DEFAULT)
    logits = logits + b_ref[...]
    m = jnp.max(logits, axis=1, keepdims=True)
    e = jnp.exp(logits - m)
    s = jnp.sum(e, axis=1, keepdims=True)
    w = e / s
    w_ref[...] = w
    ids = jax.lax.broadcasted_iota(jnp.int32, (TM, N_EXP), 1)
    m1 = jnp.max(w, axis=1, keepdims=True)
    i1 = jnp.min(jnp.where(w == m1, ids, N_EXP), axis=1, keepdims=True)
    w2 = jnp.where(ids == i1, -1.0, w)
    m2 = jnp.max(w2, axis=1, keepdims=True)
    i2 = jnp.min(jnp.where(w2 == m2, ids, N_EXP), axis=1, keepdims=True)
    tw_ref[...] = jnp.concatenate([m1, m2], axis=1)
    ti_ref[...] = jnp.concatenate([i1, i2], axis=1)


def kernel(x, W, b):
    Wt = W.T
    b2 = b.reshape(1, N_EXP)
    tw, ti, w = pl.pallas_call(
        _gate_body,
        grid=(N_TOK // TM,),
        in_specs=[
            pl.BlockSpec((TM, D_MODEL), lambda i: (i, 0)),
            pl.BlockSpec((D_MODEL, N_EXP), lambda i: (0, 0)),
            pl.BlockSpec((1, N_EXP), lambda i: (0, 0)),
        ],
        out_specs=[
            pl.BlockSpec((TM, TOP_K), lambda i: (i, 0)),
            pl.BlockSpec((TM, TOP_K), lambda i: (i, 0)),
            pl.BlockSpec((TM, N_EXP), lambda i: (i, 0)),
        ],
        out_shape=[
            jax.ShapeDtypeStruct((N_TOK, TOP_K), jnp.float32),
            jax.ShapeDtypeStruct((N_TOK, TOP_K), jnp.int32),
            jax.ShapeDtypeStruct((N_TOK, N_EXP), jnp.float32),
        ],
    )(x, Wt, b2)
    return (tw, ti, w)


# speedup vs baseline: 1.2788x; 1.2042x over previous
import jax
import jax.numpy as jnp
from jax.experimental import pallas as pl

N_TOK = 8192
D_MODEL = 4096
N_EXP = 64
TOP_K = 2
TM = 512


def _gate_body(x_ref, wt_ref, b_ref, tw_ref, ti_ref, w_ref):
    logits = jax.lax.dot_general(
        x_ref[...], wt_ref[...], (((1,), (0,)), ((), ())),
        preferred_element_type=jnp.float32,
        precision=jax.lax.Precision.DEFAULT)
    logits = logits + b_ref[...]
    m = jnp.max(logits, axis=1, keepdims=True)
    e = jnp.exp(logits - m)
    s = jnp.sum(e, axis=1, keepdims=True)
    w = e / s
    w_ref[...] = w
    ids = jax.lax.broadcasted_iota(jnp.int32, (TM, N_EXP), 1)
    m1 = jnp.max(w, axis=1, keepdims=True)
    i1 = jnp.min(jnp.where(w == m1, ids, N_EXP), axis=1, keepdims=True)
    w2 = jnp.where(ids == i1, -1.0, w)
    m2 = jnp.max(w2, axis=1, keepdims=True)
    i2 = jnp.min(jnp.where(w2 == m2, ids, N_EXP), axis=1, keepdims=True)
    tw_ref[...] = jnp.concatenate([m1, m2], axis=1)
    ti_ref[...] = jnp.concatenate([i1, i2], axis=1)


def kernel(x, W, b):
    Wt = W.T
    b2 = b.reshape(1, N_EXP)
    tw, ti, w = pl.pallas_call(
        _gate_body,
        grid=(N_TOK // TM,),
        in_specs=[
            pl.BlockSpec((TM, D_MODEL), lambda i: (i, 0)),
            pl.BlockSpec((D_MODEL, N_EXP), lambda i: (0, 0)),
            pl.BlockSpec((1, N_EXP), lambda i: (0, 0)),
        ],
        out_specs=[
            pl.BlockSpec((TM, TOP_K), lambda i: (i, 0)),
            pl.BlockSpec((TM, TOP_K), lambda i: (i, 0)),
            pl.BlockSpec((TM, N_EXP), lambda i: (i, 0)),
        ],
        out_shape=[
            jax.ShapeDtypeStruct((N_TOK, TOP_K), jnp.float32),
            jax.ShapeDtypeStruct((N_TOK, TOP_K), jnp.int32),
            jax.ShapeDtypeStruct((N_TOK, N_EXP), jnp.float32),
        ],
    )(x, Wt, b2)
    return (tw, ti, w)


# speedup vs baseline: 1.3430x; 1.0501x over previous
import jax
import jax.numpy as jnp
from jax.experimental import pallas as pl

N_TOK = 8192
D_MODEL = 4096
N_EXP = 64
TOP_K = 2
TM = 1024


def _gate_body(x_ref, wt_ref, b_ref, tw_ref, ti_ref, w_ref):
    logits = jax.lax.dot_general(
        x_ref[...], wt_ref[...], (((1,), (0,)), ((), ())),
        preferred_element_type=jnp.float32,
        precision=jax.lax.Precision.DEFAULT)
    logits = logits + b_ref[...]
    m = jnp.max(logits, axis=1, keepdims=True)
    e = jnp.exp(logits - m)
    s = jnp.sum(e, axis=1, keepdims=True)
    w = e / s
    w_ref[...] = w
    ids = jax.lax.broadcasted_iota(jnp.int32, (TM, N_EXP), 1)
    m1 = jnp.max(w, axis=1, keepdims=True)
    i1 = jnp.min(jnp.where(w == m1, ids, N_EXP), axis=1, keepdims=True)
    w2 = jnp.where(ids == i1, -1.0, w)
    m2 = jnp.max(w2, axis=1, keepdims=True)
    i2 = jnp.min(jnp.where(w2 == m2, ids, N_EXP), axis=1, keepdims=True)
    tw_ref[...] = jnp.concatenate([m1, m2], axis=1)
    ti_ref[...] = jnp.concatenate([i1, i2], axis=1)


def kernel(x, W, b):
    Wt = W.T
    b2 = b.reshape(1, N_EXP)
    tw, ti, w = pl.pallas_call(
        _gate_body,
        grid=(N_TOK // TM,),
        in_specs=[
            pl.BlockSpec((TM, D_MODEL), lambda i: (i, 0)),
            pl.BlockSpec((D_MODEL, N_EXP), lambda i: (0, 0)),
            pl.BlockSpec((1, N_EXP), lambda i: (0, 0)),
        ],
        out_specs=[
            pl.BlockSpec((TM, TOP_K), lambda i: (i, 0)),
            pl.BlockSpec((TM, TOP_K), lambda i: (i, 0)),
            pl.BlockSpec((TM, N_EXP), lambda i: (i, 0)),
        ],
        out_shape=[
            jax.ShapeDtypeStruct((N_TOK, TOP_K), jnp.float32),
            jax.ShapeDtypeStruct((N_TOK, TOP_K), jnp.int32),
            jax.ShapeDtypeStruct((N_TOK, N_EXP), jnp.float32),
        ],
    )(x, Wt, b2)
    return (tw, ti, w)
